# baseline (speedup 1.0000x reference)
import functools
import math

import jax
import jax.numpy as jnp
import numpy as np
from jax.experimental import pallas as pl
from jax.experimental.pallas import tpu as pltpu

B, L, D, H = 4, 2048, 1024, 16
DK = D // H
TOPK = int(math.log(L))
F = L // 2 + 1
FPAD = 1152

_t = np.arange(L)
_f = np.arange(FPAD)
_mask = (_f < F)[:, None]
_ang = 2.0 * np.pi * np.outer(_f, _t) / L
CF_MAT = (np.cos(_ang) * _mask).astype(np.float32)
SF_MAT = (np.sin(_ang) * _mask).astype(np.float32)
_af = np.where((_f == 0) | (_f == F - 1), 1.0, 2.0) * (_f < F) / L
_angi = 2.0 * np.pi * np.outer(_t, _f) / L
CI_MAT = (np.cos(_angi) * _af).astype(np.float32)
SI_MAT = (-np.sin(_angi) * _af).astype(np.float32)

RT = 256
FT = 192
TT = 256
VT = 256


def _k1_body(x_ref, w_ref, b_ref, o_ref):
    o_ref[0] = (
        jnp.dot(x_ref[...].astype(jnp.bfloat16), w_ref[0].astype(jnp.bfloat16), preferred_element_type=jnp.float32)
        + b_ref[0]
    )


def _k2_body(q_ref, k_ref, cf_ref, sf_ref, pr_ref, pi_ref):
    q = q_ref[0, 0].astype(jnp.bfloat16)
    k = k_ref[0, 0].astype(jnp.bfloat16)
    cf = cf_ref[...].astype(jnp.bfloat16)
    sf = sf_ref[...].astype(jnp.bfloat16)
    qr = jnp.dot(cf, q, preferred_element_type=jnp.float32)
    qi = -jnp.dot(sf, q, preferred_element_type=jnp.float32)
    kr = jnp.dot(cf, k, preferred_element_type=jnp.float32)
    ki = -jnp.dot(sf, k, preferred_element_type=jnp.float32)
    pr_ref[0] = qr * kr + qi * ki
    pi_ref[0] = qi * kr - qr * ki


def _k3_body(pr_ref, pi_ref, ci_ref, si_ref, corr_ref, mean_ref):
    corr = jnp.dot(ci_ref[...].astype(jnp.bfloat16), pr_ref[0].astype(jnp.bfloat16), preferred_element_type=jnp.float32)
    corr += jnp.dot(si_ref[...].astype(jnp.bfloat16), pi_ref[0].astype(jnp.bfloat16), preferred_element_type=jnp.float32)
    corr_ref[0] = corr
    mean_ref[0, 0, :] = jnp.mean(corr, axis=1)


def _k4_body(mv_ref, w_ref, idx_ref):
    mv = mv_ref[:, 0, :]
    bm = jnp.mean(mv, axis=0, keepdims=True)
    iota = jax.lax.broadcasted_iota(jnp.int32, (1, L), 1)
    col8 = jax.lax.broadcasted_iota(jnp.int32, (1, 8), 1)
    wacc = jnp.zeros((B, 8), jnp.float32)
    idx = jnp.zeros((1, 8), jnp.int32)
    for i in range(TOPK):
        cur = jnp.max(bm)
        am = jnp.min(jnp.where(bm == cur, iota, jnp.int32(L)))
        oh = iota == am
        wv = jnp.sum(jnp.where(oh, mv, 0.0), axis=1, keepdims=True)
        wacc = jnp.where(col8 == i, wv, wacc)
        idx = jnp.where(col8 == i, am, idx)
        bm = jnp.where(oh, -jnp.inf, bm)
    mask = col8 < TOPK
    z = jnp.where(mask, wacc, -jnp.inf)
    z = z - jnp.max(z, axis=1, keepdims=True)
    e = jnp.where(mask, jnp.exp(z), 0.0)
    w_ref[...] = e / jnp.sum(e, axis=1, keepdims=True)
    idx_ref[...] = idx


def _k5_body(idx_ref, w_ref, vd_ref, wo_ref, bo_ref, o_ref, buf, acc_ref,
             flat_ref, sem):
    b = pl.program_id(0)
    lt = pl.program_id(1)
    base = lt * VT

    def shifted_copy(i, slot):
        start = base + idx_ref[0, i]
        return pltpu.make_async_copy(
            vd_ref.at[b, pl.ds(start, VT), :, :], buf.at[slot], sem.at[slot]
        )

    shifted_copy(0, 0).start()
    for i in range(TOPK):
        slot = i % 2
        if i + 1 < TOPK:
            shifted_copy(i + 1, 1 - slot).start()
        shifted_copy(i, slot).wait()
        w = w_ref[b, i]
        if i == 0:
            acc_ref[...] = w * buf[slot]
        else:
            acc_ref[...] += w * buf[slot]
    for s in range(8):
        flat_ref[:, s * 128:(s + 1) * 128] = acc_ref[:, s, :]
    o_ref[0] = (
        jnp.dot(flat_ref[...].astype(jnp.bfloat16), wo_ref[...].astype(jnp.bfloat16), preferred_element_type=jnp.float32)
        + bo_ref[...]
    )


def kernel(x, Wq, bq, Wk, bk, Wv, bv, Wo, bo):
    f32 = jnp.float32
    cf = jnp.asarray(CF_MAT)
    sf = jnp.asarray(SF_MAT)
    ci = jnp.asarray(CI_MAT)
    si = jnp.asarray(SI_MAT)

    w_all = jnp.stack([Wq.T, Wk.T, Wv.T])
    b_all = jnp.stack([bq, bk, bv]).reshape(3, 1, D)
    xf = x.reshape(B * L, D)
    qkv3 = pl.pallas_call(
        _k1_body,
        grid=(B * L // RT, 3),
        in_specs=[
            pl.BlockSpec((RT, D), lambda r, c: (r, 0)),
            pl.BlockSpec((1, D, D), lambda r, c: (c, 0, 0)),
            pl.BlockSpec((1, 1, D), lambda r, c: (c, 0, 0)),
        ],
        out_specs=pl.BlockSpec((1, RT, D), lambda r, c: (c, r, 0)),
        out_shape=jax.ShapeDtypeStruct((3, B * L, D), f32),
    )(xf, w_all, b_all)
    qkv4 = qkv3.reshape(3, B, L, D)

    pr, pi = pl.pallas_call(
        _k2_body,
        grid=(B, FPAD // FT),
        in_specs=[
            pl.BlockSpec((1, 1, L, D), lambda b, ft: (0, b, 0, 0)),
            pl.BlockSpec((1, 1, L, D), lambda b, ft: (1, b, 0, 0)),
            pl.BlockSpec((FT, L), lambda b, ft: (ft, 0)),
            pl.BlockSpec((FT, L), lambda b, ft: (ft, 0)),
        ],
        out_specs=[
            pl.BlockSpec((1, FT, D), lambda b, ft: (b, ft, 0)),
            pl.BlockSpec((1, FT, D), lambda b, ft: (b, ft, 0)),
        ],
        out_shape=[
            jax.ShapeDtypeStruct((B, FPAD, D), f32),
            jax.ShapeDtypeStruct((B, FPAD, D), f32),
        ],
    )(qkv4, qkv4, cf, sf)

    corr, mean_value = pl.pallas_call(
        _k3_body,
        grid=(B, L // TT),
        in_specs=[
            pl.BlockSpec((1, FPAD, D), lambda b, t: (b, 0, 0)),
            pl.BlockSpec((1, FPAD, D), lambda b, t: (b, 0, 0)),
            pl.BlockSpec((TT, FPAD), lambda b, t: (t, 0)),
            pl.BlockSpec((TT, FPAD), lambda b, t: (t, 0)),
        ],
        out_specs=[
            pl.BlockSpec((1, TT, D), lambda b, t: (b, t, 0)),
            pl.BlockSpec((1, 1, TT), lambda b, t: (b, 0, t)),
        ],
        out_shape=[
            jax.ShapeDtypeStruct((B, L, D), f32),
            jax.ShapeDtypeStruct((B, 1, L), f32),
        ],
    )(pr, pi, ci, si)

    w_sm, idx = pl.pallas_call(
        _k4_body,
        in_specs=[pl.BlockSpec((B, 1, L), lambda: (0, 0, 0))],
        out_specs=[
            pl.BlockSpec((B, 8), lambda: (0, 0)),
            pl.BlockSpec((1, 8), lambda: (0, 0)),
        ],
        out_shape=[
            jax.ShapeDtypeStruct((B, 8), f32),
            jax.ShapeDtypeStruct((1, 8), jnp.int32),
        ],
    )(mean_value)

    vd = jnp.concatenate([qkv4[2], qkv4[2]], axis=1).reshape(B, 2 * L, 8, 128)
    out = pl.pallas_call(
        _k5_body,
        grid=(B, L // VT),
        in_specs=[
            pl.BlockSpec(memory_space=pltpu.SMEM),
            pl.BlockSpec(memory_space=pltpu.SMEM),
            pl.BlockSpec(memory_space=pl.ANY),
            pl.BlockSpec((D, D), lambda b, lt: (0, 0)),
            pl.BlockSpec((1, D), lambda b, lt: (0, 0)),
        ],
        out_specs=pl.BlockSpec((1, VT, D), lambda b, lt: (b, lt, 0)),
        out_shape=jax.ShapeDtypeStruct((B, L, D), f32),
        scratch_shapes=[
            pltpu.VMEM((2, VT, 8, 128), f32),
            pltpu.VMEM((VT, 8, 128), f32),
            pltpu.VMEM((VT, D), f32),
            pltpu.SemaphoreType.DMA((2,)),
        ],
    )(idx, w_sm, vd, Wo.T, bo.reshape(1, D))

    attn = corr.reshape(B, L, H, DK)
    return out, attn


# speedup vs baseline: 1.1708x; 1.1708x over previous
import functools
import math

import jax
import jax.numpy as jnp
import numpy as np
from jax.experimental import pallas as pl
from jax.experimental.pallas import tpu as pltpu

B, L, D, H = 4, 2048, 1024, 16
DK = D // H
TOPK = int(math.log(L))
F = L // 2 + 1
FPAD = 1152

_t = np.arange(L)
_f = np.arange(FPAD)
_mask = (_f < F)[:, None]
_ang = 2.0 * np.pi * np.outer(_f, _t) / L
CF_MAT = (np.cos(_ang) * _mask).astype(np.float32)
SF_MAT = (np.sin(_ang) * _mask).astype(np.float32)
_af = np.where((_f == 0) | (_f == F - 1), 1.0, 2.0) * (_f < F) / L
_angi = 2.0 * np.pi * np.outer(_t, _f) / L
CI_MAT = (np.cos(_angi) * _af).astype(np.float32)
SI_MAT = (-np.sin(_angi) * _af).astype(np.float32)

RT = 256
FT = 192
TT = 256
VT = 256


def _k1_body(x_ref, w_ref, b_ref, o_ref):
    x = x_ref[...]
    for c in range(3):
        o_ref[c] = (
            jnp.dot(x, w_ref[c], preferred_element_type=jnp.float32)
            + b_ref[c]
        )


def _k2_body(q_ref, k_ref, cf_ref, sf_ref, pr_ref, pi_ref):
    q = q_ref[0, 0]
    k = k_ref[0, 0]
    cf = cf_ref[...]
    sf = sf_ref[...]
    qr = jnp.dot(cf, q, preferred_element_type=jnp.float32)
    qi = -jnp.dot(sf, q, preferred_element_type=jnp.float32)
    kr = jnp.dot(cf, k, preferred_element_type=jnp.float32)
    ki = -jnp.dot(sf, k, preferred_element_type=jnp.float32)
    pr_ref[0] = qr * kr + qi * ki
    pi_ref[0] = qi * kr - qr * ki


def _k3_body(pr_ref, pi_ref, ci_ref, si_ref, corr_ref, mean_ref):
    corr = jnp.dot(ci_ref[...], pr_ref[0], preferred_element_type=jnp.float32)
    corr += jnp.dot(si_ref[...], pi_ref[0], preferred_element_type=jnp.float32)
    corr_ref[0] = corr
    mean_ref[0, 0, :] = jnp.mean(corr, axis=1)


def _k4_body(mv_ref, w_ref, idx_ref):
    mv = mv_ref[:, 0, :]
    bm = jnp.mean(mv, axis=0, keepdims=True)
    iota = jax.lax.broadcasted_iota(jnp.int32, (1, L), 1)
    col8 = jax.lax.broadcasted_iota(jnp.int32, (1, 8), 1)
    wacc = jnp.zeros((B, 8), jnp.float32)
    idx = jnp.zeros((1, 8), jnp.int32)
    for i in range(TOPK):
        cur = jnp.max(bm)
        am = jnp.min(jnp.where(bm == cur, iota, jnp.int32(L)))
        oh = iota == am
        wv = jnp.sum(jnp.where(oh, mv, 0.0), axis=1, keepdims=True)
        wacc = jnp.where(col8 == i, wv, wacc)
        idx = jnp.where(col8 == i, am, idx)
        bm = jnp.where(oh, -jnp.inf, bm)
    mask = col8 < TOPK
    z = jnp.where(mask, wacc, -jnp.inf)
    z = z - jnp.max(z, axis=1, keepdims=True)
    e = jnp.where(mask, jnp.exp(z), 0.0)
    w_ref[...] = e / jnp.sum(e, axis=1, keepdims=True)
    idx_ref[...] = idx


def _k5_body(idx_ref, w_ref, vd_ref, wo_ref, bo_ref, o_ref, buf, acc_ref,
             flat_ref, sem):
    b = pl.program_id(0)
    lt = pl.program_id(1)
    base = lt * VT

    def shifted_copy(i, slot):
        start = base + idx_ref[0, i]
        return pltpu.make_async_copy(
            vd_ref.at[b, pl.ds(start, VT), :, :], buf.at[slot], sem.at[slot]
        )

    shifted_copy(0, 0).start()
    for i in range(TOPK):
        slot = i % 2
        if i + 1 < TOPK:
            shifted_copy(i + 1, 1 - slot).start()
        shifted_copy(i, slot).wait()
        w = w_ref[b, i]
        if i == 0:
            acc_ref[...] = w * buf[slot]
        else:
            acc_ref[...] += w * buf[slot]
    for s in range(8):
        flat_ref[:, s * 128:(s + 1) * 128] = acc_ref[:, s, :]
    o_ref[0] = (
        jnp.dot(flat_ref[...], wo_ref[...], preferred_element_type=jnp.float32)
        + bo_ref[...]
    )


def kernel(x, Wq, bq, Wk, bk, Wv, bv, Wo, bo):
    f32 = jnp.float32
    cf = jnp.asarray(CF_MAT)
    sf = jnp.asarray(SF_MAT)
    ci = jnp.asarray(CI_MAT)
    si = jnp.asarray(SI_MAT)

    w_all = jnp.stack([Wq.T, Wk.T, Wv.T])
    b_all = jnp.stack([bq, bk, bv]).reshape(3, 1, D)
    xf = x.reshape(B * L, D)
    qkv3 = pl.pallas_call(
        _k1_body,
        grid=(B * L // RT,),
        in_specs=[
            pl.BlockSpec((RT, D), lambda r: (r, 0)),
            pl.BlockSpec((3, D, D), lambda r: (0, 0, 0)),
            pl.BlockSpec((3, 1, D), lambda r: (0, 0, 0)),
        ],
        out_specs=pl.BlockSpec((3, RT, D), lambda r: (0, r, 0)),
        out_shape=jax.ShapeDtypeStruct((3, B * L, D), f32),
    )(xf, w_all, b_all)
    qkv4 = qkv3.reshape(3, B, L, D)

    pr, pi = pl.pallas_call(
        _k2_body,
        grid=(B, FPAD // FT),
        in_specs=[
            pl.BlockSpec((1, 1, L, D), lambda b, ft: (0, b, 0, 0)),
            pl.BlockSpec((1, 1, L, D), lambda b, ft: (1, b, 0, 0)),
            pl.BlockSpec((FT, L), lambda b, ft: (ft, 0)),
            pl.BlockSpec((FT, L), lambda b, ft: (ft, 0)),
        ],
        out_specs=[
            pl.BlockSpec((1, FT, D), lambda b, ft: (b, ft, 0)),
            pl.BlockSpec((1, FT, D), lambda b, ft: (b, ft, 0)),
        ],
        out_shape=[
            jax.ShapeDtypeStruct((B, FPAD, D), f32),
            jax.ShapeDtypeStruct((B, FPAD, D), f32),
        ],
    )(qkv4, qkv4, cf, sf)

    corr, mean_value = pl.pallas_call(
        _k3_body,
        grid=(B, L // TT),
        in_specs=[
            pl.BlockSpec((1, FPAD, D), lambda b, t: (b, 0, 0)),
            pl.BlockSpec((1, FPAD, D), lambda b, t: (b, 0, 0)),
            pl.BlockSpec((TT, FPAD), lambda b, t: (t, 0)),
            pl.BlockSpec((TT, FPAD), lambda b, t: (t, 0)),
        ],
        out_specs=[
            pl.BlockSpec((1, TT, D), lambda b, t: (b, t, 0)),
            pl.BlockSpec((1, 1, TT), lambda b, t: (b, 0, t)),
        ],
        out_shape=[
            jax.ShapeDtypeStruct((B, L, D), f32),
            jax.ShapeDtypeStruct((B, 1, L), f32),
        ],
    )(pr, pi, ci, si)

    w_sm, idx = pl.pallas_call(
        _k4_body,
        in_specs=[pl.BlockSpec((B, 1, L), lambda: (0, 0, 0))],
        out_specs=[
            pl.BlockSpec((B, 8), lambda: (0, 0)),
            pl.BlockSpec((1, 8), lambda: (0, 0)),
        ],
        out_shape=[
            jax.ShapeDtypeStruct((B, 8), f32),
            jax.ShapeDtypeStruct((1, 8), jnp.int32),
        ],
    )(mean_value)

    vd = jnp.concatenate([qkv4[2], qkv4[2]], axis=1).reshape(B, 2 * L, 8, 128)
    out = pl.pallas_call(
        _k5_body,
        grid=(B, L // VT),
        in_specs=[
            pl.BlockSpec(memory_space=pltpu.SMEM),
            pl.BlockSpec(memory_space=pltpu.SMEM),
            pl.BlockSpec(memory_space=pl.ANY),
            pl.BlockSpec((D, D), lambda b, lt: (0, 0)),
            pl.BlockSpec((1, D), lambda b, lt: (0, 0)),
        ],
        out_specs=pl.BlockSpec((1, VT, D), lambda b, lt: (b, lt, 0)),
        out_shape=jax.ShapeDtypeStruct((B, L, D), f32),
        scratch_shapes=[
            pltpu.VMEM((2, VT, 8, 128), f32),
            pltpu.VMEM((VT, 8, 128), f32),
            pltpu.VMEM((VT, D), f32),
            pltpu.SemaphoreType.DMA((2,)),
        ],
    )(idx, w_sm, vd, Wo.T, bo.reshape(1, D))

    attn = corr.reshape(B, L, H, DK)
    return out, attn


# speedup vs baseline: 1.7941x; 1.5324x over previous
import math

import jax
import jax.numpy as jnp
import numpy as np
from jax.experimental import pallas as pl
from jax.experimental.pallas import tpu as pltpu

B, L, D, H = 4, 2048, 1024, 16
DK = D // H
TOPK = int(math.log(L))
F = L // 2 + 1
FPAD = 1152

_t = np.arange(L)
_f = np.arange(FPAD)
_mask = (_f < F)[:, None]
_ang = 2.0 * np.pi * np.outer(_f, _t) / L
CF_MAT = (np.cos(_ang) * _mask).astype(np.float32)
SF_MAT = (np.sin(_ang) * _mask).astype(np.float32)
_af = np.where((_f == 0) | (_f == F - 1), 1.0, 2.0) * (_f < F) / L
_angi = 2.0 * np.pi * np.outer(_t, _f) / L
CI_MAT = (np.cos(_angi) * _af).astype(np.float32)
SI_MAT = (-np.sin(_angi) * _af).astype(np.float32)

RT = 256
FT = 192
TT = 256
VT = 256
NMT = L // VT


def _k1_body(x_ref, w_ref, b_ref, qk_ref, v_ref):
    x = x_ref[...]
    for c in range(2):
        qk_ref[c] = (
            jnp.dot(x, w_ref[c], preferred_element_type=jnp.float32)
            + b_ref[c]
        )
    v = jnp.dot(x, w_ref[2], preferred_element_type=jnp.float32) + b_ref[2]
    for s in range(8):
        v_ref[0, :, s, :] = v[:, s * 128:(s + 1) * 128]


def _k2_body(q_ref, k_ref, cf_ref, sf_ref, pr_ref, pi_ref):
    q = q_ref[0, 0]
    k = k_ref[0, 0]
    cf = cf_ref[...]
    sf = sf_ref[...]
    qr = jnp.dot(cf, q, preferred_element_type=jnp.float32)
    qi = -jnp.dot(sf, q, preferred_element_type=jnp.float32)
    kr = jnp.dot(cf, k, preferred_element_type=jnp.float32)
    ki = -jnp.dot(sf, k, preferred_element_type=jnp.float32)
    pr_ref[0] = qr * kr + qi * ki
    pi_ref[0] = qi * kr - qr * ki


def _k3_body(pr_ref, pi_ref, ci_ref, si_ref, corr_ref, mean_ref):
    corr = jnp.dot(ci_ref[...], pr_ref[0], preferred_element_type=jnp.float32)
    corr += jnp.dot(si_ref[...], pi_ref[0], preferred_element_type=jnp.float32)
    corr_ref[0] = corr
    mean_ref[0, 0, :] = jnp.mean(corr, axis=1)


def _k4_body(mv_ref, w_ref, idx_ref):
    mv = mv_ref[:, 0, :]
    bm = jnp.mean(mv, axis=0, keepdims=True)
    iota = jax.lax.broadcasted_iota(jnp.int32, (1, L), 1)
    col8 = jax.lax.broadcasted_iota(jnp.int32, (1, 8), 1)
    wacc = jnp.zeros((B, 8), jnp.float32)
    idx = jnp.zeros((1, 8), jnp.int32)
    for i in range(TOPK):
        cur = jnp.max(bm)
        am = jnp.min(jnp.where(bm == cur, iota, jnp.int32(L)))
        oh = iota == am
        wv = jnp.sum(jnp.where(oh, mv, 0.0), axis=1, keepdims=True)
        wacc = jnp.where(col8 == i, wv, wacc)
        idx = jnp.where(col8 == i, am, idx)
        bm = jnp.where(oh, -jnp.inf, bm)
    mask = col8 < TOPK
    z = jnp.where(mask, wacc, -jnp.inf)
    z = z - jnp.max(z, axis=1, keepdims=True)
    e = jnp.where(mask, jnp.exp(z), 0.0)
    w_ref[...] = e / jnp.sum(e, axis=1, keepdims=True)
    idx_ref[...] = idx


def _k5_body(idx_ref, w_ref, v_ref, wo_ref, bo_ref, o_ref,
             vdbl, delays, flat_ref):
    b = pl.program_id(0)
    p = pl.program_id(1)

    @pl.when(p == 0)
    def _():
        vdbl[0:L] = v_ref[0]
        vdbl[L:2 * L] = v_ref[0]
        for i in range(TOPK):
            w = w_ref[b, i]
            s0 = idx_ref[0, i]
            slab = vdbl[pl.ds(s0, L), :, :]
            if i == 0:
                delays[...] = w * slab
            else:
                delays[...] += w * slab

    @pl.when(p > 0)
    def _():
        mt = p - 1
        for s in range(8):
            flat_ref[:, s * 128:(s + 1) * 128] = \
                delays[pl.ds(mt * VT, VT), s, :]
        o_ref[0] = (
            jnp.dot(flat_ref[...], wo_ref[...],
                    preferred_element_type=jnp.float32)
            + bo_ref[...]
        )


def kernel(x, Wq, bq, Wk, bk, Wv, bv, Wo, bo):
    f32 = jnp.float32
    cf = jnp.asarray(CF_MAT)
    sf = jnp.asarray(SF_MAT)
    ci = jnp.asarray(CI_MAT)
    si = jnp.asarray(SI_MAT)

    w_all = jnp.stack([Wq.T, Wk.T, Wv.T])
    b_all = jnp.stack([bq, bk, bv]).reshape(3, 1, D)
    xf = x.reshape(B * L, D)
    qk, vslab = pl.pallas_call(
        _k1_body,
        grid=(B * L // RT,),
        in_specs=[
            pl.BlockSpec((RT, D), lambda r: (r, 0)),
            pl.BlockSpec((3, D, D), lambda r: (0, 0, 0)),
            pl.BlockSpec((3, 1, D), lambda r: (0, 0, 0)),
        ],
        out_specs=[
            pl.BlockSpec((2, RT, D), lambda r: (0, r, 0)),
            pl.BlockSpec((1, RT, 8, 128), lambda r: (r // 8, r % 8, 0, 0)),
        ],
        out_shape=[
            jax.ShapeDtypeStruct((2, B * L, D), f32),
            jax.ShapeDtypeStruct((B, L, 8, 128), f32),
        ],
    )(xf, w_all, b_all)
    qk4 = qk.reshape(2, B, L, D)

    pr, pi = pl.pallas_call(
        _k2_body,
        grid=(B, FPAD // FT),
        in_specs=[
            pl.BlockSpec((1, 1, L, D), lambda b, ft: (0, b, 0, 0)),
            pl.BlockSpec((1, 1, L, D), lambda b, ft: (1, b, 0, 0)),
            pl.BlockSpec((FT, L), lambda b, ft: (ft, 0)),
            pl.BlockSpec((FT, L), lambda b, ft: (ft, 0)),
        ],
        out_specs=[
            pl.BlockSpec((1, FT, D), lambda b, ft: (b, ft, 0)),
            pl.BlockSpec((1, FT, D), lambda b, ft: (b, ft, 0)),
        ],
        out_shape=[
            jax.ShapeDtypeStruct((B, FPAD, D), f32),
            jax.ShapeDtypeStruct((B, FPAD, D), f32),
        ],
    )(qk4, qk4, cf, sf)

    corr, mean_value = pl.pallas_call(
        _k3_body,
        grid=(B, L // TT),
        in_specs=[
            pl.BlockSpec((1, FPAD, D), lambda b, t: (b, 0, 0)),
            pl.BlockSpec((1, FPAD, D), lambda b, t: (b, 0, 0)),
            pl.BlockSpec((TT, FPAD), lambda b, t: (t, 0)),
            pl.BlockSpec((TT, FPAD), lambda b, t: (t, 0)),
        ],
        out_specs=[
            pl.BlockSpec((1, TT, D), lambda b, t: (b, t, 0)),
            pl.BlockSpec((1, 1, TT), lambda b, t: (b, 0, t)),
        ],
        out_shape=[
            jax.ShapeDtypeStruct((B, L, D), f32),
            jax.ShapeDtypeStruct((B, 1, L), f32),
        ],
    )(pr, pi, ci, si)

    w_sm, idx = pl.pallas_call(
        _k4_body,
        in_specs=[pl.BlockSpec((B, 1, L), lambda: (0, 0, 0))],
        out_specs=[
            pl.BlockSpec((B, 8), lambda: (0, 0)),
            pl.BlockSpec((1, 8), lambda: (0, 0)),
        ],
        out_shape=[
            jax.ShapeDtypeStruct((B, 8), f32),
            jax.ShapeDtypeStruct((1, 8), jnp.int32),
        ],
    )(mean_value)

    out = pl.pallas_call(
        _k5_body,
        grid=(B, 1 + NMT),
        in_specs=[
            pl.BlockSpec(memory_space=pltpu.SMEM),
            pl.BlockSpec(memory_space=pltpu.SMEM),
            pl.BlockSpec((1, L, 8, 128), lambda b, p: (b, 0, 0, 0)),
            pl.BlockSpec((D, D), lambda b, p: (0, 0)),
            pl.BlockSpec((1, D), lambda b, p: (0, 0)),
        ],
        out_specs=pl.BlockSpec(
            (1, VT, D), lambda b, p: (b, jnp.maximum(p - 1, 0), 0)
        ),
        out_shape=jax.ShapeDtypeStruct((B, L, D), f32),
        scratch_shapes=[
            pltpu.VMEM((2 * L, 8, 128), f32),
            pltpu.VMEM((L, 8, 128), f32),
            pltpu.VMEM((VT, D), f32),
        ],
    )(idx, w_sm, vslab, Wo.T, bo.reshape(1, D))

    attn = corr.reshape(B, L, H, DK)
    return out, attn
